# baseline (speedup 1.0000x reference)
import functools

import jax
import jax.numpy as jnp
import numpy as np
from jax.experimental import pallas as pl
from jax.experimental.pallas import tpu as pltpu

_EMBED_DIM = 128
_DIM1 = _EMBED_DIM // 3
_DIM2 = _EMBED_DIM - 2 * _DIM1
_HALF = _EMBED_DIM // 2


def _chn_emb_body(mus_ref, t_ref, r_ref, o_ref, out_ref):
    mus = mus_ref[...]
    R, C = mus.shape

    neg = mus < 0.0
    idxs = jnp.clip(jnp.floor(-mus - 1.0), 0.0, 11.0)
    s = jnp.where(neg, -idxs - 1.0, jnp.floor(mus))
    s_b = jnp.broadcast_to(s[:, :, None], (R, C, _EMBED_DIM))

    d = jax.lax.broadcasted_iota(jnp.int32, (1, 1, _EMBED_DIM), 2)
    dm = (d % _HALF).astype(jnp.float32)
    omega_t = jnp.exp(dm * jnp.float32(-np.log(10000.0) / _HALF)
                      + jnp.float32(-np.log(2.0 * np.pi)))
    phase_t = jnp.where(d >= _HALF, jnp.float32(0.25), jnp.float32(0.0))

    y0 = s_b * omega_t + phase_t
    y = y0 - jnp.floor(y0 + jnp.float32(0.5))
    y2 = y * y
    p = jnp.float32(32.782657623291016)
    p = p * y2 + jnp.float32(-74.47864532470703)
    p = p * y2 + jnp.float32(81.3669204711914)
    p = p * y2 + jnp.float32(-41.33122253417969)
    p = p * y2 + jnp.float32(6.283055782318115)
    opt_val = p * y

    t0 = t_ref[0]
    t1 = t_ref[1]
    r0 = r_ref[0]
    r1 = r_ref[1]
    v0 = t0 + r0
    v1 = t0 + r1
    v2 = t1 + r1
    v3 = t1 + r0
    c1 = (-11.0 * v0 + 18.0 * v1 - 9.0 * v2 + 2.0 * v3) * jnp.float32(1.0 / 6.0)
    c2 = (2.0 * v0 - 5.0 * v1 + 4.0 * v2 - v3) * jnp.float32(0.5)
    c3 = (-v0 + 3.0 * v1 - 3.0 * v2 + v3) * jnp.float32(1.0 / 6.0)
    o0 = o_ref[0]
    o1 = o_ref[1]
    w0 = (o0 + o1) * 0.5
    g1 = (-3.0 * w0 + 4.0 * o0 - o1) * jnp.float32(0.5)
    g2 = (w0 - 2.0 * o0 + o1) * jnp.float32(0.5)

    idx_b = jnp.float32(-1.0) - s_b
    q = jnp.floor(idx_b * jnp.float32(0.25))
    rm = idx_b - 4.0 * q
    tr = ((c3 * rm + c2) * rm + c1) * rm + v0
    orb = (g2 * q + g1) * q + w0
    sar_val = tr + orb

    out_ref[...] = jnp.where(s_b < 0.0, sar_val, opt_val)


def _chn_emb(mus, t_pad, r_pad, o_pad, block_r):
    B, C = mus.shape
    grid = (B // block_r,)
    out = pl.pallas_call(
        _chn_emb_body,
        grid=grid,
        in_specs=[
            pl.BlockSpec((block_r, C), lambda i: (i, 0)),
            pl.BlockSpec((2, _EMBED_DIM), lambda i: (0, 0)),
            pl.BlockSpec((2, _EMBED_DIM), lambda i: (0, 0)),
            pl.BlockSpec((2, _EMBED_DIM), lambda i: (0, 0)),
        ],
        out_specs=pl.BlockSpec((block_r, C, _EMBED_DIM), lambda i: (i, 0, 0)),
        out_shape=jax.ShapeDtypeStruct((B, C, _EMBED_DIM), jnp.float32),
        compiler_params=pltpu.CompilerParams(
            dimension_semantics=("arbitrary",),
        ),
    )(mus, t_pad, r_pad, o_pad)
    return out


def kernel(input, embed_transmit, embed_receive, embed_orbit):
    B, C = input.shape
    CP = 56
    mus_p = jnp.pad(input, ((0, 0), (0, CP - C)))
    t_pad = jnp.pad(embed_transmit, ((0, 0), (0, _EMBED_DIM - _DIM1)))
    r_pad = jnp.pad(embed_receive, ((0, 0), (_DIM1, _DIM2)))
    o_pad = jnp.pad(embed_orbit, ((0, 0), (2 * _DIM1, 0)))
    out = _chn_emb(mus_p, t_pad, r_pad, o_pad, block_r=256)
    return jax.lax.slice(out, (0, 0, 0), (B, C, _EMBED_DIM))


# speedup vs baseline: 1.2293x; 1.2293x over previous
import functools

import jax
import jax.numpy as jnp
import numpy as np
from jax.experimental import pallas as pl
from jax.experimental.pallas import tpu as pltpu

_EMBED_DIM = 128
_DIM1 = _EMBED_DIM // 3
_DIM2 = _EMBED_DIM - 2 * _DIM1
_HALF = _EMBED_DIM // 2


def _chn_emb_body(mus_ref, t_ref, r_ref, o_ref, out_ref):
    mus = mus_ref[...]
    R, C = mus.shape

    neg = mus < 0.0
    idxs = jnp.clip(jnp.floor(-mus - 1.0), 0.0, 11.0)
    s = jnp.where(neg, -idxs - 1.0, jnp.floor(mus))
    qs = jnp.floor(idxs * jnp.float32(0.25))
    rms = idxs - 4.0 * qs
    shape3 = (R, C, _EMBED_DIM)
    s_b = jnp.broadcast_to(s[:, :, None], shape3)
    rm = jnp.broadcast_to(rms[:, :, None], shape3)
    q = jnp.broadcast_to(qs[:, :, None], shape3)

    d = jax.lax.broadcasted_iota(jnp.int32, (1, 1, _EMBED_DIM), 2)
    dm = (d % _HALF).astype(jnp.float32)
    omega_t = jnp.exp(dm * jnp.float32(-np.log(10000.0) / _HALF)
                      + jnp.float32(-np.log(2.0 * np.pi)))
    phase_t = jnp.where(d >= _HALF, jnp.float32(0.25), jnp.float32(0.0))

    y0 = s_b * omega_t + phase_t
    y = y0 - jnp.round(y0)
    y2 = y * y
    p = jnp.float32(-56.08879852294922)
    p = p * y2 + jnp.float32(77.93126678466797)
    p = p * y2 + jnp.float32(-41.09383010864258)
    p = p * y2 + jnp.float32(6.278637886047363)
    opt_val = p * y

    t0 = t_ref[0]
    t1 = t_ref[1]
    r0 = r_ref[0]
    r1 = r_ref[1]
    v0 = t0 + r0
    v1 = t0 + r1
    v2 = t1 + r1
    v3 = t1 + r0
    c1 = (-11.0 * v0 + 18.0 * v1 - 9.0 * v2 + 2.0 * v3) * jnp.float32(1.0 / 6.0)
    c2 = (2.0 * v0 - 5.0 * v1 + 4.0 * v2 - v3) * jnp.float32(0.5)
    c3 = (-v0 + 3.0 * v1 - 3.0 * v2 + v3) * jnp.float32(1.0 / 6.0)
    o0 = o_ref[0]
    o1 = o_ref[1]
    w0 = (o0 + o1) * 0.5
    g1 = (-3.0 * w0 + 4.0 * o0 - o1) * jnp.float32(0.5)
    g2 = (w0 - 2.0 * o0 + o1) * jnp.float32(0.5)

    tr = ((c3 * rm + c2) * rm + c1) * rm + v0
    orb = (g2 * q + g1) * q + w0
    sar_val = tr + orb

    out_ref[...] = jnp.where(s_b < 0.0, sar_val, opt_val)


def _chn_emb(mus, t_pad, r_pad, o_pad, block_r):
    B, C = mus.shape
    grid = (B // block_r,)
    out = pl.pallas_call(
        _chn_emb_body,
        grid=grid,
        in_specs=[
            pl.BlockSpec((block_r, C), lambda i: (i, 0)),
            pl.BlockSpec((2, _EMBED_DIM), lambda i: (0, 0)),
            pl.BlockSpec((2, _EMBED_DIM), lambda i: (0, 0)),
            pl.BlockSpec((2, _EMBED_DIM), lambda i: (0, 0)),
        ],
        out_specs=pl.BlockSpec((block_r, C, _EMBED_DIM), lambda i: (i, 0, 0)),
        out_shape=jax.ShapeDtypeStruct((B, C, _EMBED_DIM), jnp.float32),
        compiler_params=pltpu.CompilerParams(
            dimension_semantics=("arbitrary",),
        ),
    )(mus, t_pad, r_pad, o_pad)
    return out


def kernel(input, embed_transmit, embed_receive, embed_orbit):
    t_pad = jnp.pad(embed_transmit, ((0, 0), (0, _EMBED_DIM - _DIM1)))
    r_pad = jnp.pad(embed_receive, ((0, 0), (_DIM1, _DIM2)))
    o_pad = jnp.pad(embed_orbit, ((0, 0), (2 * _DIM1, 0)))
    return _chn_emb(input, t_pad, r_pad, o_pad, block_r=256)


# speedup vs baseline: 1.2794x; 1.0407x over previous
import functools

import jax
import jax.numpy as jnp
import numpy as np
from jax.experimental import pallas as pl
from jax.experimental.pallas import tpu as pltpu

_EMBED_DIM = 128
_DIM1 = _EMBED_DIM // 3
_DIM2 = _EMBED_DIM - 2 * _DIM1
_HALF = _EMBED_DIM // 2


def _chn_emb_body(mus_ref, t_ref, r_ref, o_ref, out_ref):
    mus = mus_ref[...]
    R, C = mus.shape

    neg = mus < 0.0
    idxs = jnp.clip(jnp.floor(-mus - 1.0), 0.0, 11.0)
    s = jnp.where(neg, -idxs - 1.0, jnp.floor(mus))
    qs = jnp.floor(idxs * jnp.float32(0.25))
    rms = idxs - 4.0 * qs
    shape3 = (R, C, _EMBED_DIM)
    s_b = jnp.broadcast_to(s[:, :, None], shape3)
    rm = jnp.broadcast_to(rms[:, :, None], shape3)
    q = jnp.broadcast_to(qs[:, :, None], shape3)

    d = jax.lax.broadcasted_iota(jnp.int32, (1, 1, _EMBED_DIM), 2)
    dm = (d % _HALF).astype(jnp.float32)
    omega_t = jnp.exp(dm * jnp.float32(-np.log(10000.0) / _HALF)
                      + jnp.float32(-np.log(2.0 * np.pi)))
    phase_t = jnp.where(d >= _HALF, jnp.float32(0.25), jnp.float32(0.0))

    y0 = s_b * omega_t + phase_t
    y = y0 - jnp.round(y0)
    y2 = y * y
    p = jnp.float32(-56.08879852294922)
    p = p * y2 + jnp.float32(77.93126678466797)
    p = p * y2 + jnp.float32(-41.09383010864258)
    p = p * y2 + jnp.float32(6.278637886047363)
    opt_val = p * y

    t0 = t_ref[0]
    t1 = t_ref[1]
    r0 = r_ref[0]
    r1 = r_ref[1]
    v0 = t0 + r0
    v1 = t0 + r1
    v2 = t1 + r1
    v3 = t1 + r0
    c1 = (-11.0 * v0 + 18.0 * v1 - 9.0 * v2 + 2.0 * v3) * jnp.float32(1.0 / 6.0)
    c2 = (2.0 * v0 - 5.0 * v1 + 4.0 * v2 - v3) * jnp.float32(0.5)
    c3 = (-v0 + 3.0 * v1 - 3.0 * v2 + v3) * jnp.float32(1.0 / 6.0)
    o0 = o_ref[0]
    o1 = o_ref[1]
    w0 = (o0 + o1) * 0.5
    g1 = (-3.0 * w0 + 4.0 * o0 - o1) * jnp.float32(0.5)
    g2 = (w0 - 2.0 * o0 + o1) * jnp.float32(0.5)

    w = jnp.where(d < 2 * _DIM1, rm, q)
    cc2 = c2 + g2
    cc1 = c1 + g1
    cc0 = v0 + w0
    sar_val = ((c3 * w + cc2) * w + cc1) * w + cc0

    out_ref[...] = jnp.where(s_b < 0.0, sar_val, opt_val)


def _chn_emb(mus, t_pad, r_pad, o_pad, block_r):
    B, C = mus.shape
    grid = (B // block_r,)
    out = pl.pallas_call(
        _chn_emb_body,
        grid=grid,
        in_specs=[
            pl.BlockSpec((block_r, C), lambda i: (i, 0)),
            pl.BlockSpec((2, _EMBED_DIM), lambda i: (0, 0)),
            pl.BlockSpec((2, _EMBED_DIM), lambda i: (0, 0)),
            pl.BlockSpec((2, _EMBED_DIM), lambda i: (0, 0)),
        ],
        out_specs=pl.BlockSpec((block_r, C, _EMBED_DIM), lambda i: (i, 0, 0)),
        out_shape=jax.ShapeDtypeStruct((B, C, _EMBED_DIM), jnp.float32),
        compiler_params=pltpu.CompilerParams(
            dimension_semantics=("arbitrary",),
        ),
    )(mus, t_pad, r_pad, o_pad)
    return out


def kernel(input, embed_transmit, embed_receive, embed_orbit):
    t_pad = jnp.pad(embed_transmit, ((0, 0), (0, _EMBED_DIM - _DIM1)))
    r_pad = jnp.pad(embed_receive, ((0, 0), (_DIM1, _DIM2)))
    o_pad = jnp.pad(embed_orbit, ((0, 0), (2 * _DIM1, 0)))
    return _chn_emb(input, t_pad, r_pad, o_pad, block_r=256)


# speedup vs baseline: 1.3944x; 1.0899x over previous
import functools

import jax
import jax.numpy as jnp
import numpy as np
from jax.experimental import pallas as pl
from jax.experimental.pallas import tpu as pltpu

_EMBED_DIM = 128
_DIM1 = _EMBED_DIM // 3
_DIM2 = _EMBED_DIM - 2 * _DIM1
_HALF = _EMBED_DIM // 2


def _chn_emb_body(mus_ref, t_ref, r_ref, o_ref, out_ref):
    mus = mus_ref[...]
    R, C = mus.shape

    neg = mus < 0.0
    idxs = jnp.clip(jnp.floor(-mus - 1.0), 0.0, 11.0)
    s = jnp.where(neg, -idxs - 1.0, jnp.floor(mus))
    qs = jnp.floor(idxs * jnp.float32(0.25))
    rms = idxs - 4.0 * qs
    shape3 = (R, C, _EMBED_DIM)
    s_b = jnp.broadcast_to(s[:, :, None], shape3)
    rm = jnp.broadcast_to(rms[:, :, None], shape3)
    q = jnp.broadcast_to(qs[:, :, None], shape3)

    d = jax.lax.broadcasted_iota(jnp.int32, (1, 1, _EMBED_DIM), 2)
    dm = (d % _HALF).astype(jnp.float32)
    omega_t = jnp.exp(dm * jnp.float32(-np.log(10000.0) / _HALF)
                      + jnp.float32(-np.log(2.0 * np.pi)))
    phase_t = jnp.where(d >= _HALF, jnp.float32(0.25), jnp.float32(0.0))

    y0 = s_b * omega_t + phase_t
    y = y0 - jnp.round(y0)
    y2 = y * y
    p = jnp.float32(-56.08879852294922)
    p = p * y2 + jnp.float32(77.93126678466797)
    p = p * y2 + jnp.float32(-41.09383010864258)
    p = p * y2 + jnp.float32(6.278637886047363)
    opt_val = p * y

    t0 = t_ref[0]
    t1 = t_ref[1]
    r0 = r_ref[0]
    r1 = r_ref[1]
    v0 = t0 + r0
    v1 = t0 + r1
    v2 = t1 + r1
    v3 = t1 + r0
    c1 = (-11.0 * v0 + 18.0 * v1 - 9.0 * v2 + 2.0 * v3) * jnp.float32(1.0 / 6.0)
    c2 = (2.0 * v0 - 5.0 * v1 + 4.0 * v2 - v3) * jnp.float32(0.5)
    c3 = (-v0 + 3.0 * v1 - 3.0 * v2 + v3) * jnp.float32(1.0 / 6.0)
    o0 = o_ref[0]
    o1 = o_ref[1]
    w0 = (o0 + o1) * 0.5
    g1 = (-3.0 * w0 + 4.0 * o0 - o1) * jnp.float32(0.5)
    g2 = (w0 - 2.0 * o0 + o1) * jnp.float32(0.5)

    w = jnp.where(d < 2 * _DIM1, rm, q)
    cc2 = c2 + g2
    cc1 = c1 + g1
    cc0 = v0 + w0
    sar_val = ((c3 * w + cc2) * w + cc1) * w + cc0

    out_ref[...] = jnp.where(s_b < 0.0, sar_val, opt_val).astype(out_ref.dtype)


def _chn_emb(mus, t_pad, r_pad, o_pad, block_r):
    B, C = mus.shape
    grid = (B // block_r,)
    out = pl.pallas_call(
        _chn_emb_body,
        grid=grid,
        in_specs=[
            pl.BlockSpec((block_r, C), lambda i: (i, 0)),
            pl.BlockSpec((2, _EMBED_DIM), lambda i: (0, 0)),
            pl.BlockSpec((2, _EMBED_DIM), lambda i: (0, 0)),
            pl.BlockSpec((2, _EMBED_DIM), lambda i: (0, 0)),
        ],
        out_specs=pl.BlockSpec((block_r, C, _EMBED_DIM), lambda i: (i, 0, 0)),
        out_shape=jax.ShapeDtypeStruct((B, C, _EMBED_DIM), jnp.bfloat16),
        compiler_params=pltpu.CompilerParams(
            dimension_semantics=("arbitrary",),
        ),
    )(mus, t_pad, r_pad, o_pad)
    return out


def kernel(input, embed_transmit, embed_receive, embed_orbit):
    t_pad = jnp.pad(embed_transmit, ((0, 0), (0, _EMBED_DIM - _DIM1)))
    r_pad = jnp.pad(embed_receive, ((0, 0), (_DIM1, _DIM2)))
    o_pad = jnp.pad(embed_orbit, ((0, 0), (2 * _DIM1, 0)))
    return _chn_emb(input, t_pad, r_pad, o_pad, block_r=256).astype(jnp.float32)


# speedup vs baseline: 1.3972x; 1.0020x over previous
import functools

import jax
import jax.numpy as jnp
import numpy as np
from jax.experimental import pallas as pl
from jax.experimental.pallas import tpu as pltpu

_EMBED_DIM = 128
_DIM1 = _EMBED_DIM // 3
_DIM2 = _EMBED_DIM - 2 * _DIM1
_HALF = _EMBED_DIM // 2


def _chn_emb_body(mus_ref, t_ref, r_ref, o_ref, out_ref):
    mus = mus_ref[...]
    R, C = mus.shape

    neg = mus < 0.0
    idxs = jnp.clip(jnp.floor(-mus - 1.0), 0.0, 11.0)
    s = jnp.where(neg, -idxs - 1.0, jnp.floor(mus))
    qs = jnp.floor(idxs * jnp.float32(0.25))
    rms = idxs - 4.0 * qs
    shape3 = (R, C, _EMBED_DIM)
    s_b = jnp.broadcast_to(s[:, :, None], shape3)
    rm = jnp.broadcast_to(rms[:, :, None], shape3)
    q = jnp.broadcast_to(qs[:, :, None], shape3)

    d = jax.lax.broadcasted_iota(jnp.int32, (1, 1, _EMBED_DIM), 2)
    dm = (d % _HALF).astype(jnp.float32)
    omega_t = jnp.exp(dm * jnp.float32(-np.log(10000.0) / _HALF)
                      + jnp.float32(-np.log(2.0 * np.pi)))
    phase_t = jnp.where(d >= _HALF, jnp.float32(0.25), jnp.float32(0.0))

    y0 = s_b * omega_t + phase_t
    y = y0 - jnp.round(y0)
    y2 = y * y
    p = jnp.float32(-56.08879852294922)
    p = p * y2 + jnp.float32(77.93126678466797)
    p = p * y2 + jnp.float32(-41.09383010864258)
    p = p * y2 + jnp.float32(6.278637886047363)
    opt_val = p * y

    t0 = t_ref[0]
    t1 = t_ref[1]
    r0 = r_ref[0]
    r1 = r_ref[1]
    v0 = t0 + r0
    v1 = t0 + r1
    v2 = t1 + r1
    v3 = t1 + r0
    c1 = (-11.0 * v0 + 18.0 * v1 - 9.0 * v2 + 2.0 * v3) * jnp.float32(1.0 / 6.0)
    c2 = (2.0 * v0 - 5.0 * v1 + 4.0 * v2 - v3) * jnp.float32(0.5)
    c3 = (-v0 + 3.0 * v1 - 3.0 * v2 + v3) * jnp.float32(1.0 / 6.0)
    o0 = o_ref[0]
    o1 = o_ref[1]
    w0 = (o0 + o1) * 0.5
    g1 = (-3.0 * w0 + 4.0 * o0 - o1) * jnp.float32(0.5)
    g2 = (w0 - 2.0 * o0 + o1) * jnp.float32(0.5)

    w = jnp.where(d < 2 * _DIM1, rm, q)
    cc2 = c2 + g2
    cc1 = c1 + g1
    cc0 = v0 + w0
    sar_val = ((c3 * w + cc2) * w + cc1) * w + cc0

    out_ref[...] = jnp.where(s_b < 0.0, sar_val, opt_val).astype(out_ref.dtype)


def _chn_emb(mus, t_pad, r_pad, o_pad, block_r):
    B, C = mus.shape
    grid = (B // block_r,)
    out = pl.pallas_call(
        _chn_emb_body,
        grid=grid,
        in_specs=[
            pl.BlockSpec((block_r, C), lambda i: (i, 0)),
            pl.BlockSpec((2, _EMBED_DIM), lambda i: (0, 0)),
            pl.BlockSpec((2, _EMBED_DIM), lambda i: (0, 0)),
            pl.BlockSpec((2, _EMBED_DIM), lambda i: (0, 0)),
        ],
        out_specs=pl.BlockSpec((block_r, C, _EMBED_DIM), lambda i: (i, 0, 0)),
        out_shape=jax.ShapeDtypeStruct((B, C, _EMBED_DIM), jnp.bfloat16),
        compiler_params=pltpu.CompilerParams(
            dimension_semantics=("arbitrary",),
        ),
    )(mus, t_pad, r_pad, o_pad)
    return out


def kernel(input, embed_transmit, embed_receive, embed_orbit):
    t_pad = jnp.pad(embed_transmit, ((0, 0), (0, _EMBED_DIM - _DIM1)))
    r_pad = jnp.pad(embed_receive, ((0, 0), (_DIM1, _DIM2)))
    o_pad = jnp.pad(embed_orbit, ((0, 0), (2 * _DIM1, 0)))
    return _chn_emb(input, t_pad, r_pad, o_pad, block_r=512).astype(jnp.float32)
